# baseline (speedup 1.0000x reference)
import jax
import jax.numpy as jnp
from jax import lax
from jax.experimental import pallas as pl
from jax.experimental.pallas import tpu as pltpu
from jax.experimental.pallas import tpu_sc as plsc

B, S, D = 1, 2048, 1024
H = 2048
D_OUT = 1024
E = 8
T = B * S
TOP_K = 2
A = TOP_K * T
TM = 256
NT = A // TM + E
NP = NT * TM

NW = 32
CHS = T // (NW // 2)
CHG = 64


def _router_body(x_ref, rw_ref, rb_ref, d1_ref, d2_ref, g1_ref, g2_ref,
                 te_ref, rbi_ref):
    x = x_ref[...]
    rw = rw_ref[...]
    logits = jax.lax.dot_general(
        x, rw, (((1,), (1,)), ((), ())),
        preferred_element_type=jnp.float32) + rb_ref[...]
    iota = lax.broadcasted_iota(jnp.int32, (T, E), 1)
    m1 = jnp.max(logits, axis=1, keepdims=True)
    a1 = jnp.min(jnp.where(logits == m1, iota, E), axis=1, keepdims=True)
    masked = jnp.where(iota == a1, -jnp.inf, logits)
    m2 = jnp.max(masked, axis=1, keepdims=True)
    a2 = jnp.min(jnp.where(masked == m2, iota, E), axis=1, keepdims=True)
    d = jnp.exp(m2 - m1)
    g1_ref[...] = 1.0 / (1.0 + d)
    g2_ref[...] = d / (1.0 + d)

    oh1 = (iota == a1).astype(jnp.float32)
    oh2 = (iota == a2).astype(jnp.float32)
    r_i = lax.broadcasted_iota(jnp.int32, (128, 128), 0)
    c_i = lax.broadcasted_iota(jnp.int32, (128, 128), 1)
    ls128 = (c_i < r_i).astype(jnp.bfloat16)

    def excl_counts(oh):
        prefix = jnp.zeros((1, E), jnp.float32)
        outs = []
        for c in range(T // 128):
            oh_c = oh[c * 128:(c + 1) * 128, :]
            within = jax.lax.dot_general(
                ls128, oh_c.astype(jnp.bfloat16), (((1,), (0,)), ((), ())),
                preferred_element_type=jnp.float32)
            outs.append(within + prefix)
            prefix = prefix + jnp.sum(oh_c, axis=0, keepdims=True)
        return jnp.concatenate(outs, axis=0), prefix

    pcs1, cnt1 = excl_counts(oh1)
    pcs2, cnt2 = excl_counts(oh2)
    rank1 = jnp.sum(jnp.where(iota == a1, pcs1, 0.0), axis=1, keepdims=True)
    rank2 = (jnp.sum(jnp.where(iota == a2, cnt1 + pcs2, 0.0), axis=1,
                     keepdims=True))
    cnt = cnt1 + cnt2
    ptiles = jnp.floor((cnt + (TM - 1)) * (1.0 / TM))
    e_i = lax.broadcasted_iota(jnp.int32, (E, E), 0)
    f_i = lax.broadcasted_iota(jnp.int32, (E, E), 1)
    tri_incl = (e_i <= f_i).astype(jnp.float32)
    cum_pt = jax.lax.dot_general(
        ptiles, tri_incl, (((1,), (0,)), ((), ())),
        precision=jax.lax.Precision.HIGHEST,
        preferred_element_type=jnp.float32)
    poff = (cum_pt - ptiles) * float(TM)
    d1_ref[...] = (jnp.sum(jnp.where(iota == a1, poff, 0.0), axis=1,
                           keepdims=True) + rank1).astype(jnp.int32)
    d2_ref[...] = (jnp.sum(jnp.where(iota == a2, poff, 0.0), axis=1,
                           keepdims=True) + rank2).astype(jnp.int32)

    n_act = cum_pt[0, E - 1].astype(jnp.int32)
    e_last = jnp.int32(0)
    for e in range(E):
        e_last = jnp.where(cnt[0, e] > 0.0, jnp.int32(e), e_last)
    i128 = lax.broadcasted_iota(jnp.int32, (1, 128), 1)
    raw = jnp.zeros((1, 128), jnp.int32)
    for e in range(E):
        raw = raw + (i128 >= cum_pt[0, e].astype(jnp.int32)).astype(jnp.int32)
    te_ref[...] = jnp.minimum(raw, e_last)
    rbi_ref[...] = jnp.minimum(i128, n_act - 1)


TH = 512
NH = H // TH


def _ffn_body(te_ref, rbi_ref, xs_ref, w1_ref, b1_ref, w2_ref, b2_ref,
              os_ref):
    i = pl.program_id(0)
    j = pl.program_id(1)

    @pl.when(i <= rbi_ref[127])
    def _():
        xrow = xs_ref[...].astype(jnp.bfloat16)
        h = jax.lax.dot_general(
            xrow, w1_ref[0].astype(jnp.bfloat16), (((1,), (1,)), ((), ())),
            preferred_element_type=jnp.float32)
        h = jnp.maximum(h + b1_ref[0], 0.0)
        contrib = jax.lax.dot_general(
            h.astype(jnp.bfloat16), w2_ref[0].astype(jnp.bfloat16),
            (((1,), (1,)), ((), ())),
            preferred_element_type=jnp.float32)

        @pl.when(j == 0)
        def _():
            os_ref[...] = contrib + b2_ref[0]

        @pl.when(j != 0)
        def _():
            os_ref[...] = os_ref[...] + contrib


def _combine_body(a_ref, b_ref, g1_ref, g2_ref, out_ref):
    out_ref[...] = g1_ref[...] * a_ref[...] + g2_ref[...] * b_ref[...]


def kernel(x, router_w, router_b, W1, b1, W2, b2):
    x_flat = x.reshape(T, D)
    rb2 = router_b.reshape(1, E)
    d1, d2, g1, g2, te, rbi = pl.pallas_call(
        _router_body,
        out_shape=(
            jax.ShapeDtypeStruct((T, 1), jnp.int32),
            jax.ShapeDtypeStruct((T, 1), jnp.int32),
            jax.ShapeDtypeStruct((T, 1), jnp.float32),
            jax.ShapeDtypeStruct((T, 1), jnp.float32),
            jax.ShapeDtypeStruct((1, 128), jnp.int32),
            jax.ShapeDtypeStruct((1, 128), jnp.int32),
        ),
        in_specs=[
            pl.BlockSpec((T, D), lambda: (0, 0)),
            pl.BlockSpec((E, D), lambda: (0, 0)),
            pl.BlockSpec((1, E), lambda: (0, 0)),
        ],
        out_specs=(
            pl.BlockSpec((T, 1), lambda: (0, 0)),
            pl.BlockSpec((T, 1), lambda: (0, 0)),
            pl.BlockSpec((T, 1), lambda: (0, 0)),
            pl.BlockSpec((T, 1), lambda: (0, 0)),
            pl.BlockSpec((1, 128), lambda: (0, 0)),
            pl.BlockSpec((1, 128), lambda: (0, 0)),
        ),
    )(x_flat, router_w, rb2)

    d1f = d1.reshape(T)
    d2f = d2.reshape(T)
    te_arr = te.reshape(128)
    rbi_arr = rbi.reshape(128)

    mesh = plsc.VectorSubcoreMesh(core_axis_name="c", subcore_axis_name="s")

    def sc_scatter(x_a, d1_a, d2_a):
        def body(x_hbm, d1_hbm, d2_hbm, xs_hbm, idx_v, rows_v):
            wid = lax.axis_index("s") * 2 + lax.axis_index("c")
            half = wid // 16
            base = (wid % 16) * CHS
            for hf in range(CHS // CHG):
                b0 = base + hf * CHG

                @pl.when(half == 0)
                def _():
                    pltpu.sync_copy(d1_hbm.at[pl.ds(b0, CHG)], idx_v)

                @pl.when(half == 1)
                def _():
                    pltpu.sync_copy(d2_hbm.at[pl.ds(b0, CHG)], idx_v)

                pltpu.sync_copy(x_hbm.at[pl.ds(b0, CHG)], rows_v)
                pltpu.sync_copy(rows_v, xs_hbm.at[idx_v])

        return pl.kernel(
            body,
            out_type=jax.ShapeDtypeStruct((NP, D), jnp.float32),
            mesh=mesh,
            scratch_types=[
                pltpu.VMEM((CHG,), jnp.int32),
                pltpu.VMEM((CHG, D), jnp.float32),
            ],
        )(x_a, d1_a, d2_a)

    xsb = sc_scatter(x_flat, d1f, d2f)

    b1r = b1.reshape(E, 1, H)
    b2r = b2.reshape(E, 1, D_OUT)
    os_out = pl.pallas_call(
        _ffn_body,
        grid_spec=pltpu.PrefetchScalarGridSpec(
            num_scalar_prefetch=2,
            grid=(NT, NH),
            in_specs=[
                pl.BlockSpec((TM, D), lambda i, j, te, rbi: (rbi[i], 0)),
                pl.BlockSpec((1, TH, D), lambda i, j, te, rbi: (te[i], j, 0)),
                pl.BlockSpec((1, 1, TH), lambda i, j, te, rbi: (te[i], 0, j)),
                pl.BlockSpec((1, D_OUT, TH),
                             lambda i, j, te, rbi: (te[i], 0, j)),
                pl.BlockSpec((1, 1, D_OUT),
                             lambda i, j, te, rbi: (te[i], 0, 0)),
            ],
            out_specs=pl.BlockSpec((TM, D_OUT),
                                   lambda i, j, te, rbi: (rbi[i], 0)),
        ),
        out_shape=jax.ShapeDtypeStruct((NP, D_OUT), jnp.float32),
        compiler_params=pltpu.CompilerParams(
            dimension_semantics=("arbitrary", "arbitrary")),
    )(te_arr, rbi_arr, xsb, W1, b1r, W2, b2r)

    def sc_gather(os_a, d1_a, d2_a):
        def body(os_hbm, d1_hbm, d2_hbm, gt_hbm, idx_v, rows_v):
            wid = lax.axis_index("s") * 2 + lax.axis_index("c")
            half = wid // 16
            base = (wid % 16) * CHS
            for hf in range(CHS // CHG):
                b0 = base + hf * CHG

                @pl.when(half == 0)
                def _():
                    pltpu.sync_copy(d1_hbm.at[pl.ds(b0, CHG)], idx_v)

                @pl.when(half == 1)
                def _():
                    pltpu.sync_copy(d2_hbm.at[pl.ds(b0, CHG)], idx_v)

                pltpu.sync_copy(os_hbm.at[idx_v], rows_v)
                pltpu.sync_copy(rows_v,
                                gt_hbm.at[pl.ds(half * T + b0, CHG)])

        return pl.kernel(
            body,
            out_type=jax.ShapeDtypeStruct((A, D_OUT), jnp.float32),
            mesh=mesh,
            scratch_types=[
                pltpu.VMEM((CHG,), jnp.int32),
                pltpu.VMEM((CHG, D_OUT), jnp.float32),
            ],
        )(os_a, d1_a, d2_a)

    gt = sc_gather(os_out, d1f, d2f)

    out = pl.pallas_call(
        _combine_body,
        grid=(T // TM,),
        in_specs=[
            pl.BlockSpec((TM, D_OUT), lambda i: (i, 0)),
            pl.BlockSpec((TM, D_OUT), lambda i: (i + T // TM, 0)),
            pl.BlockSpec((TM, 1), lambda i: (i, 0)),
            pl.BlockSpec((TM, 1), lambda i: (i, 0)),
        ],
        out_specs=pl.BlockSpec((TM, D_OUT), lambda i: (i, 0)),
        out_shape=jax.ShapeDtypeStruct((T, D_OUT), jnp.float32),
    )(gt, gt, g1, g2)
    return out.reshape(B, S, D_OUT)


# speedup vs baseline: 1.6391x; 1.6391x over previous
import jax
import jax.numpy as jnp
from jax import lax
from jax.experimental import pallas as pl
from jax.experimental.pallas import tpu as pltpu
from jax.experimental.pallas import tpu_sc as plsc

B, S, D = 1, 2048, 1024
H = 2048
D_OUT = 1024
E = 8
T = B * S
TOP_K = 2
A = TOP_K * T
TM = 256
NT = A // TM + E
NP = NT * TM
DW = D // 2
OW = D_OUT // 2

NW = 32
CHS = T // (NW // 2)


def _pack16(a_f32, b_f32):
    ai = lax.bitcast_convert_type(a_f32, jnp.int32)
    bi = lax.bitcast_convert_type(b_f32, jnp.int32)
    a16 = ((ai + 0x7FFF + ((ai >> 16) & 1)) >> 16) & 0xFFFF
    b16 = ((bi + 0x7FFF + ((bi >> 16) & 1)) >> 16) & 0xFFFF
    return a16 | (b16 << 16)


def _unpack16(w_i32):
    a = lax.bitcast_convert_type(w_i32 << 16, jnp.float32)
    b = lax.bitcast_convert_type(w_i32 & jnp.int32(-65536), jnp.float32)
    return a, b


def _router_body(x_ref, rw_ref, rb_ref, d1_ref, d2_ref, g1_ref, g2_ref,
                 te_ref, rbi_ref, xp_ref):
    x = x_ref[...]
    rw = rw_ref[...]
    xp_ref[...] = _pack16(x[:, :DW], x[:, DW:])
    logits = jax.lax.dot_general(
        x, rw, (((1,), (1,)), ((), ())),
        preferred_element_type=jnp.float32) + rb_ref[...]
    iota = lax.broadcasted_iota(jnp.int32, (T, E), 1)
    m1 = jnp.max(logits, axis=1, keepdims=True)
    a1 = jnp.min(jnp.where(logits == m1, iota, E), axis=1, keepdims=True)
    masked = jnp.where(iota == a1, -jnp.inf, logits)
    m2 = jnp.max(masked, axis=1, keepdims=True)
    a2 = jnp.min(jnp.where(masked == m2, iota, E), axis=1, keepdims=True)
    d = jnp.exp(m2 - m1)
    g1_ref[...] = 1.0 / (1.0 + d)
    g2_ref[...] = d / (1.0 + d)

    oh1 = (iota == a1).astype(jnp.float32)
    oh2 = (iota == a2).astype(jnp.float32)
    r_i = lax.broadcasted_iota(jnp.int32, (128, 128), 0)
    c_i = lax.broadcasted_iota(jnp.int32, (128, 128), 1)
    ls128 = (c_i < r_i).astype(jnp.bfloat16)

    def excl_counts(oh):
        prefix = jnp.zeros((1, E), jnp.float32)
        outs = []
        for c in range(T // 128):
            oh_c = oh[c * 128:(c + 1) * 128, :]
            within = jax.lax.dot_general(
                ls128, oh_c.astype(jnp.bfloat16), (((1,), (0,)), ((), ())),
                preferred_element_type=jnp.float32)
            outs.append(within + prefix)
            prefix = prefix + jnp.sum(oh_c, axis=0, keepdims=True)
        return jnp.concatenate(outs, axis=0), prefix

    pcs1, cnt1 = excl_counts(oh1)
    pcs2, cnt2 = excl_counts(oh2)
    rank1 = jnp.sum(jnp.where(iota == a1, pcs1, 0.0), axis=1, keepdims=True)
    rank2 = (jnp.sum(jnp.where(iota == a2, cnt1 + pcs2, 0.0), axis=1,
                     keepdims=True))
    cnt = cnt1 + cnt2
    ptiles = jnp.floor((cnt + (TM - 1)) * (1.0 / TM))
    e_i = lax.broadcasted_iota(jnp.int32, (E, E), 0)
    f_i = lax.broadcasted_iota(jnp.int32, (E, E), 1)
    tri_incl = (e_i <= f_i).astype(jnp.float32)
    cum_pt = jax.lax.dot_general(
        ptiles, tri_incl, (((1,), (0,)), ((), ())),
        precision=jax.lax.Precision.HIGHEST,
        preferred_element_type=jnp.float32)
    poff = (cum_pt - ptiles) * float(TM)
    d1_ref[...] = (jnp.sum(jnp.where(iota == a1, poff, 0.0), axis=1,
                           keepdims=True) + rank1).astype(jnp.int32)
    d2_ref[...] = (jnp.sum(jnp.where(iota == a2, poff, 0.0), axis=1,
                           keepdims=True) + rank2).astype(jnp.int32)

    n_act = cum_pt[0, E - 1].astype(jnp.int32)
    e_last = jnp.int32(0)
    for e in range(E):
        e_last = jnp.where(cnt[0, e] > 0.0, jnp.int32(e), e_last)
    i128 = lax.broadcasted_iota(jnp.int32, (1, 128), 1)
    raw = jnp.zeros((1, 128), jnp.int32)
    for e in range(E):
        raw = raw + (i128 >= cum_pt[0, e].astype(jnp.int32)).astype(jnp.int32)
    te_ref[...] = jnp.minimum(raw, e_last)
    rbi_ref[...] = jnp.minimum(i128, n_act - 1)


def _ffn_body(te_ref, rbi_ref, xs_ref, w1_ref, b1_ref, w2_ref, b2_ref,
              os_ref):
    i = pl.program_id(0)

    @pl.when(i <= rbi_ref[127])
    def _():
        x_lo, x_hi = _unpack16(xs_ref[...])
        w1 = w1_ref[0]
        h = jax.lax.dot_general(
            x_lo.astype(jnp.bfloat16), w1[:, :DW].astype(jnp.bfloat16),
            (((1,), (1,)), ((), ())),
            preferred_element_type=jnp.float32)
        h = h + jax.lax.dot_general(
            x_hi.astype(jnp.bfloat16), w1[:, DW:].astype(jnp.bfloat16),
            (((1,), (1,)), ((), ())),
            preferred_element_type=jnp.float32)
        h = jnp.maximum(h + b1_ref[0], 0.0)
        o = jax.lax.dot_general(
            h.astype(jnp.bfloat16), w2_ref[0].astype(jnp.bfloat16),
            (((1,), (1,)), ((), ())),
            preferred_element_type=jnp.float32) + b2_ref[0]
        os_ref[...] = _pack16(o[:, :OW], o[:, OW:])


def _combine_body(a_ref, b_ref, g1_ref, g2_ref, out_ref):
    a_lo, a_hi = _unpack16(a_ref[...])
    b_lo, b_hi = _unpack16(b_ref[...])
    g1v = g1_ref[...]
    g2v = g2_ref[...]
    out_ref[:, :OW] = g1v * a_lo + g2v * b_lo
    out_ref[:, OW:] = g1v * a_hi + g2v * b_hi


def kernel(x, router_w, router_b, W1, b1, W2, b2):
    x_flat = x.reshape(T, D)
    rb2 = router_b.reshape(1, E)
    d1, d2, g1, g2, te, rbi, xp = pl.pallas_call(
        _router_body,
        out_shape=(
            jax.ShapeDtypeStruct((T, 1), jnp.int32),
            jax.ShapeDtypeStruct((T, 1), jnp.int32),
            jax.ShapeDtypeStruct((T, 1), jnp.float32),
            jax.ShapeDtypeStruct((T, 1), jnp.float32),
            jax.ShapeDtypeStruct((1, 128), jnp.int32),
            jax.ShapeDtypeStruct((1, 128), jnp.int32),
            jax.ShapeDtypeStruct((T, DW), jnp.int32),
        ),
        in_specs=[
            pl.BlockSpec((T, D), lambda: (0, 0)),
            pl.BlockSpec((E, D), lambda: (0, 0)),
            pl.BlockSpec((1, E), lambda: (0, 0)),
        ],
        out_specs=(
            pl.BlockSpec((T, 1), lambda: (0, 0)),
            pl.BlockSpec((T, 1), lambda: (0, 0)),
            pl.BlockSpec((T, 1), lambda: (0, 0)),
            pl.BlockSpec((T, 1), lambda: (0, 0)),
            pl.BlockSpec((1, 128), lambda: (0, 0)),
            pl.BlockSpec((1, 128), lambda: (0, 0)),
            pl.BlockSpec((T, DW), lambda: (0, 0)),
        ),
    )(x_flat, router_w, rb2)

    d1f = d1.reshape(T)
    d2f = d2.reshape(T)
    te_arr = te.reshape(128)
    rbi_arr = rbi.reshape(128)

    mesh = plsc.VectorSubcoreMesh(core_axis_name="c", subcore_axis_name="s")

    def sc_scatter(x_a, d1_a, d2_a):
        def body(x_hbm, d1_hbm, d2_hbm, xs_hbm, idx_v, rows_v):
            wid = lax.axis_index("s") * 2 + lax.axis_index("c")
            half = wid // 16
            base = (wid % 16) * CHS

            @pl.when(half == 0)
            def _():
                pltpu.sync_copy(d1_hbm.at[pl.ds(base, CHS)], idx_v)

            @pl.when(half == 1)
            def _():
                pltpu.sync_copy(d2_hbm.at[pl.ds(base, CHS)], idx_v)

            pltpu.sync_copy(x_hbm.at[pl.ds(base, CHS)], rows_v)
            pltpu.sync_copy(rows_v, xs_hbm.at[idx_v])

        return pl.kernel(
            body,
            out_type=jax.ShapeDtypeStruct((NP, DW), jnp.int32),
            mesh=mesh,
            scratch_types=[
                pltpu.VMEM((CHS,), jnp.int32),
                pltpu.VMEM((CHS, DW), jnp.int32),
            ],
        )(x_a, d1_a, d2_a)

    xsp = sc_scatter(xp, d1f, d2f)

    b1r = b1.reshape(E, 1, H)
    b2r = b2.reshape(E, 1, D_OUT)
    os_out = pl.pallas_call(
        _ffn_body,
        grid_spec=pltpu.PrefetchScalarGridSpec(
            num_scalar_prefetch=2,
            grid=(NT,),
            in_specs=[
                pl.BlockSpec((TM, DW), lambda i, te, rbi: (rbi[i], 0)),
                pl.BlockSpec((1, H, D), lambda i, te, rbi: (te[i], 0, 0)),
                pl.BlockSpec((1, 1, H), lambda i, te, rbi: (te[i], 0, 0)),
                pl.BlockSpec((1, D_OUT, H), lambda i, te, rbi: (te[i], 0, 0)),
                pl.BlockSpec((1, 1, D_OUT), lambda i, te, rbi: (te[i], 0, 0)),
            ],
            out_specs=pl.BlockSpec((TM, OW), lambda i, te, rbi: (rbi[i], 0)),
        ),
        out_shape=jax.ShapeDtypeStruct((NP, OW), jnp.int32),
        compiler_params=pltpu.CompilerParams(
            dimension_semantics=("arbitrary",)),
    )(te_arr, rbi_arr, xsp, W1, b1r, W2, b2r)

    def sc_gather(os_a, d1_a, d2_a):
        def body(os_hbm, d1_hbm, d2_hbm, gt_hbm, idx_v, rows_v):
            wid = lax.axis_index("s") * 2 + lax.axis_index("c")
            half = wid // 16
            base = (wid % 16) * CHS

            @pl.when(half == 0)
            def _():
                pltpu.sync_copy(d1_hbm.at[pl.ds(base, CHS)], idx_v)

            @pl.when(half == 1)
            def _():
                pltpu.sync_copy(d2_hbm.at[pl.ds(base, CHS)], idx_v)

            pltpu.sync_copy(os_hbm.at[idx_v], rows_v)
            pltpu.sync_copy(rows_v, gt_hbm.at[pl.ds(half * T + base, CHS)])

        return pl.kernel(
            body,
            out_type=jax.ShapeDtypeStruct((A, OW), jnp.int32),
            mesh=mesh,
            scratch_types=[
                pltpu.VMEM((CHS,), jnp.int32),
                pltpu.VMEM((CHS, OW), jnp.int32),
            ],
        )(os_a, d1_a, d2_a)

    gt = sc_gather(os_out, d1f, d2f)

    out = pl.pallas_call(
        _combine_body,
        grid=(T // TM,),
        in_specs=[
            pl.BlockSpec((TM, OW), lambda i: (i, 0)),
            pl.BlockSpec((TM, OW), lambda i: (i + T // TM, 0)),
            pl.BlockSpec((TM, 1), lambda i: (i, 0)),
            pl.BlockSpec((TM, 1), lambda i: (i, 0)),
        ],
        out_specs=pl.BlockSpec((TM, D_OUT), lambda i: (i, 0)),
        out_shape=jax.ShapeDtypeStruct((T, D_OUT), jnp.float32),
    )(gt, gt, g1, g2)
    return out.reshape(B, S, D_OUT)
